# baseline (speedup 1.0000x reference)
import functools

import jax
import jax.numpy as jnp
from jax.experimental import pallas as pl
from jax.experimental.pallas import tpu as pltpu

_CHUNK = 512
_NBUF = 8


def _router_kernel(x_hbm, m_ref, wt_ref, b_ref,
                   logits_ref, tki_ref, tkw_ref, aux_ref,
                   bufs, sems, *, n_chunks, target_load):
    def start_copy(j, slot):
        pltpu.make_async_copy(
            x_hbm.at[pl.ds(j * _CHUNK, _CHUNK), :],
            bufs.at[slot],
            sems.at[slot],
        ).start()

    def wait_copy(j, slot):
        pltpu.make_async_copy(
            x_hbm.at[pl.ds(j * _CHUNK, _CHUNK), :],
            bufs.at[slot],
            sems.at[slot],
        ).wait()

    for k in range(min(_NBUF, n_chunks)):
        start_copy(k, k)

    def body(j, acc):
        slot = jax.lax.rem(j, _NBUF)
        wait_copy(j, slot)
        x = bufs[slot]
        logits = jax.lax.dot_general(
            x, wt_ref[...], (((1,), (0,)), ((), ())),
            preferred_element_type=jnp.float32,
            precision=jax.lax.Precision.DEFAULT)

        @pl.when(j + _NBUF < n_chunks)
        def _():
            start_copy(j + _NBUF, slot)

        logits = logits + m_ref[pl.ds(j * _CHUNK, _CHUNK), :] * b_ref[...]
        logits_ref[pl.ds(j * _CHUNK, _CHUNK), :] = logits

        mx = jnp.max(logits, axis=1, keepdims=True)
        e = jnp.exp(logits - mx)
        s = jnp.sum(e, axis=1, keepdims=True)
        probs = e / s

        iota = jax.lax.broadcasted_iota(jnp.int32, probs.shape, 1)
        big = jnp.int32(2**30)
        v1 = jnp.max(probs, axis=1, keepdims=True)
        i1 = jnp.min(jnp.where(probs == v1, iota, big), axis=1, keepdims=True)
        probs2 = jnp.where(iota == i1, jnp.float32(-1.0), probs)
        v2 = jnp.max(probs2, axis=1, keepdims=True)
        i2 = jnp.min(jnp.where(probs2 == v2, iota, big), axis=1, keepdims=True)
        tkw_ref[pl.ds(j * _CHUNK, _CHUNK), :] = jnp.concatenate([v1, v2], 1)
        tki_ref[pl.ds(j * _CHUNK, _CHUNK), :] = jnp.concatenate([i1, i2], 1)

        return acc + jnp.sum(probs, axis=0, keepdims=True)

    acc0 = jnp.zeros((1, 16), jnp.float32)
    acc = jax.lax.fori_loop(0, n_chunks, body, acc0)
    aux_ref[...] = jnp.mean((acc - target_load) ** 2).reshape(1, 1)


def kernel(hidden_states, mass, W, mass_bias):
    B, T, C = hidden_states.shape
    E = W.shape[0]
    N = B * T
    x = hidden_states.reshape(N, C)
    m = mass.reshape(N, 1)
    wt = W.T
    b = mass_bias.reshape(1, E)
    n_chunks = N // _CHUNK

    kfn = functools.partial(_router_kernel, n_chunks=n_chunks,
                            target_load=float(N) / float(E))
    logits, tki, tkw, aux = pl.pallas_call(
        kfn,
        in_specs=[
            pl.BlockSpec(memory_space=pltpu.MemorySpace.HBM),
            pl.BlockSpec(memory_space=pltpu.MemorySpace.VMEM),
            pl.BlockSpec(memory_space=pltpu.MemorySpace.VMEM),
            pl.BlockSpec(memory_space=pltpu.MemorySpace.VMEM),
        ],
        out_specs=[
            pl.BlockSpec(memory_space=pltpu.MemorySpace.VMEM),
            pl.BlockSpec(memory_space=pltpu.MemorySpace.VMEM),
            pl.BlockSpec(memory_space=pltpu.MemorySpace.VMEM),
            pl.BlockSpec(memory_space=pltpu.MemorySpace.VMEM),
        ],
        out_shape=[
            jax.ShapeDtypeStruct((N, E), jnp.float32),
            jax.ShapeDtypeStruct((N, 2), jnp.int32),
            jax.ShapeDtypeStruct((N, 2), jnp.float32),
            jax.ShapeDtypeStruct((1, 1), jnp.float32),
        ],
        scratch_shapes=[
            pltpu.VMEM((_NBUF, _CHUNK, C), jnp.float32),
            pltpu.SemaphoreType.DMA((_NBUF,)),
        ],
    )(x, m, wt, b)
    return (logits, tki, aux.reshape(()), tkw)
